# baseline (speedup 1.0000x reference)
import functools

import jax
import jax.numpy as jnp
from jax import lax
from jax.experimental import pallas as pl
from jax.experimental.pallas import tpu as pltpu, tpu_sc as plsc

N = 10000
E = 320000
B = 16
AF = 108
EF = 41
H = 128
L = 4
NT = 5
GF = 239

_NSC = 2
_NTILE = 16
_NW = _NSC * _NTILE
_FPT = H // _NW
_CW = 2048
_NCH = 160
_EP = _CW * _NCH
_NP = 10240
_NBLK = 10
_BN = _NP // _NBLK

_BE = 2560


def _silu(v):
    return v * jax.nn.sigmoid(v)


def _embed_body(xt_ref, w1_ref, b1_ref, w2_ref, b2_ref, o_ref, ot_ref):
    t = _silu(lax.dot_general(xt_ref[...], w1_ref[...], (((0,), (0,)), ((), ())),
                              preferred_element_type=jnp.float32)
              + b1_ref[...])
    h = jnp.dot(t, w2_ref[...], preferred_element_type=jnp.float32) + b2_ref[...]
    o_ref[pl.ds(0, N), :] = h
    ot_ref[:, :, pl.ds(0, N)] = jnp.transpose(h).reshape(_NW, _FPT, N)


def _embed(xt, aW1, ab1, aW2, ab2):
    full = lambda shp: pl.BlockSpec(shp, lambda: tuple(0 for _ in shp))
    return pl.pallas_call(
        _embed_body,
        in_specs=[
            full((AF, N)),
            full((AF, H)),
            full((1, H)),
            full((H, H)),
            full((1, H)),
        ],
        out_specs=[full((_NP, H)), full((_NW, _FPT, _NP))],
        out_shape=[jax.ShapeDtypeStruct((_NP, H), jnp.float32),
                   jax.ShapeDtypeStruct((_NW, _FPT, _NP), jnp.float32)],
    )(xt, aW1, ab1.reshape(1, H), aW2, ab2.reshape(1, H))


def _edge_body(eat_ref, w1_ref, b1_ref, w2_ref, b2_ref, o_ref):
    tt = _silu(lax.dot_general(w1_ref[0], eat_ref[...], (((0,), (0,)), ((), ())),
                               preferred_element_type=jnp.float32)
               + b1_ref[0])
    wt = (lax.dot_general(w2_ref[0], tt, (((0,), (0,)), ((), ())),
                          preferred_element_type=jnp.float32)
          + b2_ref[0])
    o_ref[0] = wt.reshape(_NW, _FPT, _BE)


def _edge_filters(eat, eW1, eb1, eW2, eb2):
    nblk = E // _BE
    return pl.pallas_call(
        _edge_body,
        grid=(nblk, L),
        in_specs=[
            pl.BlockSpec((EF, _BE), lambda i, l: (0, i)),
            pl.BlockSpec((1, EF, H), lambda i, l: (l, 0, 0)),
            pl.BlockSpec((1, H, 1), lambda i, l: (l, 0, 0)),
            pl.BlockSpec((1, H, H), lambda i, l: (l, 0, 0)),
            pl.BlockSpec((1, H, 1), lambda i, l: (l, 0, 0)),
        ],
        out_specs=pl.BlockSpec((1, _NW, _FPT, _BE), lambda i, l: (l, 0, 0, i)),
        out_shape=jax.ShapeDtypeStruct((L, _NW, _FPT, _EP), jnp.float32),
    )(eat, eW1, eb1.reshape(L, H, 1), eW2, eb2.reshape(L, H, 1))


def _make_scatter(layer):
    mesh = plsc.VectorSubcoreMesh(core_axis_name="c", subcore_axis_name="s")

    @functools.partial(
        pl.kernel,
        out_type=jax.ShapeDtypeStruct((_NW, _FPT, _NP), jnp.float32),
        mesh=mesh,
        scratch_types=[
            pltpu.VMEM((_FPT, _NP), jnp.float32),
            pltpu.VMEM((_FPT, _NP), jnp.float32),
            pltpu.VMEM((_FPT, _CW), jnp.float32),
            pltpu.VMEM((_FPT, _CW), jnp.float32),
            pltpu.VMEM((_CW,), jnp.int32),
            pltpu.VMEM((_CW,), jnp.int32),
            pltpu.VMEM((_CW,), jnp.int32),
            pltpu.VMEM((_CW,), jnp.int32),
            pltpu.SemaphoreType.DMA,
            pltpu.SemaphoreType.DMA,
        ],
        compiler_params=pltpu.CompilerParams(needs_layout_passes=False),
    )
    def scatter(ht_hbm, wallt_hbm, src_hbm, dst_hbm, out_hbm,
                hcol, acc, w0, w1, s0, s1, d0, d1, sem0, sem1):
        c = lax.axis_index("c")
        s = lax.axis_index("s")
        t = c * _NTILE + s
        wv = [w0, w1]
        sv = [s0, s1]
        dv = [d0, d1]
        sem = [sem0, sem1]

        pltpu.sync_copy(ht_hbm.at[t], hcol)

        @plsc.parallel_loop(0, _NP // 16, 1, unroll=4)
        def zero(k):
            for f in range(_FPT):
                acc[f, pl.ds(k * 16, 16)] = jnp.zeros((16,), jnp.float32)

        def issue_loads(j, b):
            base = j * _CW
            pltpu.async_copy(wallt_hbm.at[layer, t, :, pl.ds(base, _CW)],
                             wv[b], sem[b])
            pltpu.async_copy(src_hbm.at[pl.ds(base, _CW)], sv[b], sem[b])
            pltpu.async_copy(dst_hbm.at[pl.ds(base, _CW)], dv[b], sem[b])

        fvec = [jnp.full((16,), f, jnp.int32) for f in range(_FPT)]

        def do_chunk(j, b):
            pltpu.make_async_copy(wallt_hbm.at[layer, t, :, pl.ds(0, _CW)],
                                  wv[b], sem[b]).wait()
            pltpu.make_async_copy(src_hbm.at[pl.ds(0, _CW)], sv[b], sem[b]).wait()
            pltpu.make_async_copy(dst_hbm.at[pl.ds(0, _CW)], dv[b], sem[b]).wait()

            @plsc.parallel_loop(0, _CW // 16, 1, unroll=2)
            def grp(g):
                sidx = sv[b][pl.ds(g * 16, 16)]
                didx = dv[b][pl.ds(g * 16, 16)]
                for f in range(_FPT):
                    vals = plsc.load_gather(hcol, [fvec[f], sidx])
                    w = wv[b][f, pl.ds(g * 16, 16)]
                    plsc.addupdate_scatter(acc, [fvec[f], didx], vals * w)

            @pl.when(j + 2 < _NCH)
            def _():
                issue_loads(j + 2, b)

        issue_loads(0, 0)
        issue_loads(1, 1)
        do_chunk(0, 0)
        do_chunk(1, 1)

        def round_body(r, _):
            do_chunk(2 * r, 0)
            do_chunk(2 * r + 1, 1)
            return 0
        lax.fori_loop(1, _NCH // 2, round_body, 0)

        pltpu.sync_copy(acc, out_hbm.at[t])

    return scatter


def _node_body(p_ref, h_ref, w1_ref, b1_ref, w2_ref, b2_ref, g_ref, bt_ref,
               o_ref, ot_ref):
    agg = jnp.transpose(p_ref[...].reshape(H, _BN))
    t = _silu(jnp.dot(agg, w1_ref[...], preferred_element_type=jnp.float32)
              + b1_ref[...])
    out = jnp.dot(t, w2_ref[...], preferred_element_type=jnp.float32) + b2_ref[...]
    z = h_ref[...] + out
    mu = jnp.mean(z, axis=-1, keepdims=True)
    zc = z - mu
    var = jnp.mean(zc * zc, axis=-1, keepdims=True)
    hn = zc / jnp.sqrt(var + 1e-5) * g_ref[...] + bt_ref[...]
    o_ref[...] = hn
    ot_ref[...] = jnp.transpose(hn).reshape(_NW, _FPT, _BN)


def _node(partials, h, nW1l, nb1l, nW2l, nb2l, lngl, lnbl):
    return pl.pallas_call(
        _node_body,
        grid=(_NBLK,),
        in_specs=[
            pl.BlockSpec((_NW, _FPT, _BN), lambda i: (0, 0, i)),
            pl.BlockSpec((_BN, H), lambda i: (i, 0)),
            pl.BlockSpec((H, H), lambda i: (0, 0)),
            pl.BlockSpec((1, H), lambda i: (0, 0)),
            pl.BlockSpec((H, H), lambda i: (0, 0)),
            pl.BlockSpec((1, H), lambda i: (0, 0)),
            pl.BlockSpec((1, H), lambda i: (0, 0)),
            pl.BlockSpec((1, H), lambda i: (0, 0)),
        ],
        out_specs=[pl.BlockSpec((_BN, H), lambda i: (i, 0)),
                   pl.BlockSpec((_NW, _FPT, _BN), lambda i: (0, 0, i))],
        out_shape=[jax.ShapeDtypeStruct((_NP, H), jnp.float32),
                   jax.ShapeDtypeStruct((_NW, _FPT, _NP), jnp.float32)],
    )(partials, h, nW1l, nb1l.reshape(1, H), nW2l, nb2l.reshape(1, H),
      lngl.reshape(1, H), lnbl.reshape(1, H))


def _pool_body(h_ref, b_ref, gf_ref, gW_ref, gb_ref, oW1a_ref, oW1b_ref,
               ob1_ref, oW2_ref, ob2_ref, oW3_ref, ob3_ref, o_ref,
               sums_ref, cnts_ref):
    i = pl.program_id(0)

    @pl.when(i == 0)
    def _init():
        sums_ref[...] = jnp.zeros((B, H), jnp.float32)
        cnts_ref[...] = jnp.zeros((B, H), jnp.float32)

    bvec = b_ref[...]
    oh = (bvec == lax.broadcasted_iota(jnp.int32, (_BN, B), 1)).astype(jnp.float32)
    sums_ref[...] += lax.dot_general(oh, h_ref[...], (((0,), (0,)), ((), ())),
                                     preferred_element_type=jnp.float32)
    cnts_ref[...] += lax.dot_general(oh, jnp.ones((_BN, H), jnp.float32),
                                     (((0,), (0,)), ((), ())),
                                     preferred_element_type=jnp.float32)

    @pl.when(i == _NBLK - 1)
    def _head():
        repr_ = sums_ref[...] / jnp.maximum(cnts_ref[...], 1.0)
        gp = _silu(jnp.dot(gf_ref[...], gW_ref[...],
                           preferred_element_type=jnp.float32) + gb_ref[...])
        h1 = _silu(jnp.dot(repr_, oW1a_ref[...], preferred_element_type=jnp.float32)
                   + jnp.dot(gp, oW1b_ref[...], preferred_element_type=jnp.float32)
                   + ob1_ref[...])
        h2 = _silu(jnp.dot(h1, oW2_ref[...], preferred_element_type=jnp.float32)
                   + ob2_ref[...])
        o_ref[...] = (jnp.dot(h2, oW3_ref[...], preferred_element_type=jnp.float32)
                      + ob3_ref[...])


def _pool_head(h, batch2d, gf, gW, gb, oW1, ob1, oW2, ob2, oW3, ob3):
    full = lambda shp: pl.BlockSpec(shp, lambda i: tuple(0 for _ in shp))
    return pl.pallas_call(
        _pool_body,
        grid=(_NBLK,),
        in_specs=[
            pl.BlockSpec((_BN, H), lambda i: (i, 0)),
            pl.BlockSpec((_BN, 1), lambda i: (i, 0)),
            full((B, GF)),
            full((GF, H)),
            full((1, H)),
            full((H, H)),
            full((H, H)),
            full((1, H)),
            full((H, H // 2)),
            full((1, H // 2)),
            full((H // 2, NT)),
            full((1, NT)),
        ],
        out_specs=full((B, NT)),
        out_shape=jax.ShapeDtypeStruct((B, NT), jnp.float32),
        scratch_shapes=[
            pltpu.VMEM((B, H), jnp.float32),
            pltpu.VMEM((B, H), jnp.float32),
        ],
    )(h, batch2d, gf, gW, gb.reshape(1, H), oW1[:H], oW1[H:],
      ob1.reshape(1, H), oW2, ob2.reshape(1, H // 2), oW3, ob3.reshape(1, NT))


def kernel(x, edge_index, edge_attr, global_features, batch,
           aW1, ab1, aW2, ab2,
           eW1, eb1, eW2, eb2, nW1, nb1, nW2, nb2, lng, lnb,
           gW, gb, oW1, ob1, oW2, ob2, oW3, ob3):
    pad = _EP - E
    src = jnp.concatenate([edge_index[0], jnp.zeros((pad,), jnp.int32)])
    dst = jnp.concatenate([edge_index[1], jnp.full((pad,), N, jnp.int32)])
    batchp = jnp.concatenate([batch, jnp.full((_NP - N,), B, jnp.int32)])
    h, ht = _embed(x.T, aW1, ab1, aW2, ab2)
    wallt = _edge_filters(edge_attr.T, eW1, eb1, eW2, eb2)
    for l in range(L):
        aggt = _make_scatter(l)(ht, wallt, src, dst)
        h, ht = _node(aggt, h, nW1[l], nb1[l], nW2[l], nb2[l], lng[l], lnb[l])
    return _pool_head(h, batchp.reshape(_NP, 1), global_features, gW, gb,
                      oW1, ob1, oW2, ob2, oW3, ob3)


# speedup vs baseline: 1.2456x; 1.2456x over previous
import functools

import jax
import jax.numpy as jnp
from jax import lax
from jax.experimental import pallas as pl
from jax.experimental.pallas import tpu as pltpu, tpu_sc as plsc

N = 10000
E = 320000
B = 16
AF = 108
EF = 41
H = 128
L = 4
NT = 5
GF = 239

_NSC = 2
_NTILE = 16
_NW = _NSC * _NTILE
_FPT = H // _NW
_CW = 2048
_NCH = 160
_EP = _CW * _NCH
_NP = 10240
_NBLK = 10
_BN = _NP // _NBLK

_BE = 2560


def _silu(v):
    return v * jax.nn.sigmoid(v)


def _embed_body(xt_ref, w1_ref, b1_ref, w2_ref, b2_ref, o_ref, ot_ref):
    t = _silu(lax.dot_general(xt_ref[...], w1_ref[...], (((0,), (0,)), ((), ())),
                              preferred_element_type=jnp.float32)
              + b1_ref[...])
    h = jnp.dot(t, w2_ref[...], preferred_element_type=jnp.float32) + b2_ref[...]
    o_ref[pl.ds(0, N), :] = h
    ot_ref[:, :, pl.ds(0, N)] = jnp.transpose(h).reshape(_NW, _FPT, N)


def _embed(xt, aW1, ab1, aW2, ab2):
    full = lambda shp: pl.BlockSpec(shp, lambda: tuple(0 for _ in shp))
    return pl.pallas_call(
        _embed_body,
        in_specs=[
            full((AF, N)),
            full((AF, H)),
            full((1, H)),
            full((H, H)),
            full((1, H)),
        ],
        out_specs=[full((_NP, H)), full((_NW, _FPT, _NP))],
        out_shape=[jax.ShapeDtypeStruct((_NP, H), jnp.float32),
                   jax.ShapeDtypeStruct((_NW, _FPT, _NP), jnp.float32)],
    )(xt, aW1, ab1.reshape(1, H), aW2, ab2.reshape(1, H))


def _edge_body(eat_ref, w1_ref, b1_ref, w2_ref, b2_ref, o_ref):
    tt = _silu(lax.dot_general(w1_ref[0], eat_ref[...], (((0,), (0,)), ((), ())),
                               preferred_element_type=jnp.float32)
               + b1_ref[0])
    wt = (lax.dot_general(w2_ref[0], tt, (((0,), (0,)), ((), ())),
                          preferred_element_type=jnp.float32)
          + b2_ref[0])
    o_ref[0] = wt.reshape(_NW, _FPT, _BE)


def _edge_filters(eat, eW1l, eb1l, eW2l, eb2l):
    nblk = E // _BE
    return pl.pallas_call(
        _edge_body,
        grid=(nblk,),
        in_specs=[
            pl.BlockSpec((EF, _BE), lambda i: (0, i)),
            pl.BlockSpec((1, EF, H), lambda i: (0, 0, 0)),
            pl.BlockSpec((1, H, 1), lambda i: (0, 0, 0)),
            pl.BlockSpec((1, H, H), lambda i: (0, 0, 0)),
            pl.BlockSpec((1, H, 1), lambda i: (0, 0, 0)),
        ],
        out_specs=pl.BlockSpec((1, _NW, _FPT, _BE), lambda i: (0, 0, 0, i)),
        out_shape=jax.ShapeDtypeStruct((1, _NW, _FPT, _EP), jnp.float32),
    )(eat, eW1l.reshape(1, EF, H), eb1l.reshape(1, H, 1),
      eW2l.reshape(1, H, H), eb2l.reshape(1, H, 1))


def _make_scatter(layer):
    mesh = plsc.VectorSubcoreMesh(core_axis_name="c", subcore_axis_name="s")

    @functools.partial(
        pl.kernel,
        out_type=jax.ShapeDtypeStruct((_NW, _FPT, _NP), jnp.float32),
        mesh=mesh,
        scratch_types=[
            pltpu.VMEM((_FPT, _NP), jnp.float32),
            pltpu.VMEM((_FPT, _NP), jnp.float32),
            pltpu.VMEM((_FPT, _CW), jnp.float32),
            pltpu.VMEM((_FPT, _CW), jnp.float32),
            pltpu.VMEM((_CW,), jnp.int32),
            pltpu.VMEM((_CW,), jnp.int32),
            pltpu.VMEM((_CW,), jnp.int32),
            pltpu.VMEM((_CW,), jnp.int32),
            pltpu.SemaphoreType.DMA,
            pltpu.SemaphoreType.DMA,
        ],
        compiler_params=pltpu.CompilerParams(needs_layout_passes=False),
    )
    def scatter(ht_hbm, wallt_hbm, src_hbm, dst_hbm, out_hbm,
                hcol, acc, w0, w1, s0, s1, d0, d1, sem0, sem1):
        c = lax.axis_index("c")
        s = lax.axis_index("s")
        t = c * _NTILE + s
        wv = [w0, w1]
        sv = [s0, s1]
        dv = [d0, d1]
        sem = [sem0, sem1]

        pltpu.sync_copy(ht_hbm.at[t], hcol)

        @plsc.parallel_loop(0, _NP // 16, 1, unroll=4)
        def zero(k):
            for f in range(_FPT):
                acc[f, pl.ds(k * 16, 16)] = jnp.zeros((16,), jnp.float32)

        def issue_loads(j, b):
            base = j * _CW
            pltpu.async_copy(wallt_hbm.at[0, t, :, pl.ds(base, _CW)],
                             wv[b], sem[b])
            pltpu.async_copy(src_hbm.at[pl.ds(base, _CW)], sv[b], sem[b])
            pltpu.async_copy(dst_hbm.at[pl.ds(base, _CW)], dv[b], sem[b])

        fvec = [jnp.full((16,), f, jnp.int32) for f in range(_FPT)]

        def do_chunk(j, b):
            pltpu.make_async_copy(wallt_hbm.at[0, t, :, pl.ds(0, _CW)],
                                  wv[b], sem[b]).wait()
            pltpu.make_async_copy(src_hbm.at[pl.ds(0, _CW)], sv[b], sem[b]).wait()
            pltpu.make_async_copy(dst_hbm.at[pl.ds(0, _CW)], dv[b], sem[b]).wait()

            @plsc.parallel_loop(0, _CW // 16, 1, unroll=2)
            def grp(g):
                sidx = sv[b][pl.ds(g * 16, 16)]
                didx = dv[b][pl.ds(g * 16, 16)]
                for f in range(_FPT):
                    vals = plsc.load_gather(hcol, [fvec[f], sidx])
                    w = wv[b][f, pl.ds(g * 16, 16)]
                    plsc.addupdate_scatter(acc, [fvec[f], didx], vals * w)

            @pl.when(j + 2 < _NCH)
            def _():
                issue_loads(j + 2, b)

        issue_loads(0, 0)
        issue_loads(1, 1)
        do_chunk(0, 0)
        do_chunk(1, 1)

        def round_body(r, _):
            do_chunk(2 * r, 0)
            do_chunk(2 * r + 1, 1)
            return 0
        lax.fori_loop(1, _NCH // 2, round_body, 0)

        pltpu.sync_copy(acc, out_hbm.at[t])

    return scatter


def _node_body(p_ref, h_ref, w1_ref, b1_ref, w2_ref, b2_ref, g_ref, bt_ref,
               o_ref, ot_ref):
    agg = jnp.transpose(p_ref[...].reshape(H, _BN))
    t = _silu(jnp.dot(agg, w1_ref[...], preferred_element_type=jnp.float32)
              + b1_ref[...])
    out = jnp.dot(t, w2_ref[...], preferred_element_type=jnp.float32) + b2_ref[...]
    z = h_ref[...] + out
    mu = jnp.mean(z, axis=-1, keepdims=True)
    zc = z - mu
    var = jnp.mean(zc * zc, axis=-1, keepdims=True)
    hn = zc / jnp.sqrt(var + 1e-5) * g_ref[...] + bt_ref[...]
    o_ref[...] = hn
    ot_ref[...] = jnp.transpose(hn).reshape(_NW, _FPT, _BN)


def _node(partials, h, nW1l, nb1l, nW2l, nb2l, lngl, lnbl):
    return pl.pallas_call(
        _node_body,
        grid=(_NBLK,),
        in_specs=[
            pl.BlockSpec((_NW, _FPT, _BN), lambda i: (0, 0, i)),
            pl.BlockSpec((_BN, H), lambda i: (i, 0)),
            pl.BlockSpec((H, H), lambda i: (0, 0)),
            pl.BlockSpec((1, H), lambda i: (0, 0)),
            pl.BlockSpec((H, H), lambda i: (0, 0)),
            pl.BlockSpec((1, H), lambda i: (0, 0)),
            pl.BlockSpec((1, H), lambda i: (0, 0)),
            pl.BlockSpec((1, H), lambda i: (0, 0)),
        ],
        out_specs=[pl.BlockSpec((_BN, H), lambda i: (i, 0)),
                   pl.BlockSpec((_NW, _FPT, _BN), lambda i: (0, 0, i))],
        out_shape=[jax.ShapeDtypeStruct((_NP, H), jnp.float32),
                   jax.ShapeDtypeStruct((_NW, _FPT, _NP), jnp.float32)],
    )(partials, h, nW1l, nb1l.reshape(1, H), nW2l, nb2l.reshape(1, H),
      lngl.reshape(1, H), lnbl.reshape(1, H))


def _pool_body(h_ref, b_ref, gf_ref, gW_ref, gb_ref, oW1a_ref, oW1b_ref,
               ob1_ref, oW2_ref, ob2_ref, oW3_ref, ob3_ref, o_ref,
               sums_ref, cnts_ref):
    i = pl.program_id(0)

    @pl.when(i == 0)
    def _init():
        sums_ref[...] = jnp.zeros((B, H), jnp.float32)
        cnts_ref[...] = jnp.zeros((B, H), jnp.float32)

    bvec = b_ref[...]
    oh = (bvec == lax.broadcasted_iota(jnp.int32, (_BN, B), 1)).astype(jnp.float32)
    sums_ref[...] += lax.dot_general(oh, h_ref[...], (((0,), (0,)), ((), ())),
                                     preferred_element_type=jnp.float32)
    cnts_ref[...] += lax.dot_general(oh, jnp.ones((_BN, H), jnp.float32),
                                     (((0,), (0,)), ((), ())),
                                     preferred_element_type=jnp.float32)

    @pl.when(i == _NBLK - 1)
    def _head():
        repr_ = sums_ref[...] / jnp.maximum(cnts_ref[...], 1.0)
        gp = _silu(jnp.dot(gf_ref[...], gW_ref[...],
                           preferred_element_type=jnp.float32) + gb_ref[...])
        h1 = _silu(jnp.dot(repr_, oW1a_ref[...], preferred_element_type=jnp.float32)
                   + jnp.dot(gp, oW1b_ref[...], preferred_element_type=jnp.float32)
                   + ob1_ref[...])
        h2 = _silu(jnp.dot(h1, oW2_ref[...], preferred_element_type=jnp.float32)
                   + ob2_ref[...])
        o_ref[...] = (jnp.dot(h2, oW3_ref[...], preferred_element_type=jnp.float32)
                      + ob3_ref[...])


def _pool_head(h, batch2d, gf, gW, gb, oW1, ob1, oW2, ob2, oW3, ob3):
    full = lambda shp: pl.BlockSpec(shp, lambda i: tuple(0 for _ in shp))
    return pl.pallas_call(
        _pool_body,
        grid=(_NBLK,),
        in_specs=[
            pl.BlockSpec((_BN, H), lambda i: (i, 0)),
            pl.BlockSpec((_BN, 1), lambda i: (i, 0)),
            full((B, GF)),
            full((GF, H)),
            full((1, H)),
            full((H, H)),
            full((H, H)),
            full((1, H)),
            full((H, H // 2)),
            full((1, H // 2)),
            full((H // 2, NT)),
            full((1, NT)),
        ],
        out_specs=full((B, NT)),
        out_shape=jax.ShapeDtypeStruct((B, NT), jnp.float32),
        scratch_shapes=[
            pltpu.VMEM((B, H), jnp.float32),
            pltpu.VMEM((B, H), jnp.float32),
        ],
    )(h, batch2d, gf, gW, gb.reshape(1, H), oW1[:H], oW1[H:],
      ob1.reshape(1, H), oW2, ob2.reshape(1, H // 2), oW3, ob3.reshape(1, NT))


def kernel(x, edge_index, edge_attr, global_features, batch,
           aW1, ab1, aW2, ab2,
           eW1, eb1, eW2, eb2, nW1, nb1, nW2, nb2, lng, lnb,
           gW, gb, oW1, ob1, oW2, ob2, oW3, ob3):
    pad = _EP - E
    src = jnp.concatenate([edge_index[0], jnp.zeros((pad,), jnp.int32)])
    dst = jnp.concatenate([edge_index[1], jnp.full((pad,), N, jnp.int32)])
    batchp = jnp.concatenate([batch, jnp.full((_NP - N,), B, jnp.int32)])
    h, ht = _embed(x.T, aW1, ab1, aW2, ab2)
    eat = edge_attr.T
    wallts = [_edge_filters(eat, eW1[l], eb1[l], eW2[l], eb2[l])
              for l in range(L)]
    for l in range(L):
        aggt = _make_scatter(l)(ht, wallts[l], src, dst)
        h, ht = _node(aggt, h, nW1[l], nb1[l], nW2[l], nb2[l], lng[l], lnb[l])
    return _pool_head(h, batchp.reshape(_NP, 1), global_features, gW, gb,
                      oW1, ob1, oW2, ob2, oW3, ob3)


# speedup vs baseline: 1.3209x; 1.0605x over previous
import functools

import jax
import jax.numpy as jnp
from jax import lax
from jax.experimental import pallas as pl
from jax.experimental.pallas import tpu as pltpu, tpu_sc as plsc

N = 10000
E = 320000
B = 16
AF = 108
EF = 41
H = 128
L = 4
NT = 5
GF = 239

_NSC = 2
_NTILE = 16
_NW = _NSC * _NTILE
_FPT = H // _NW
_CW = 2048
_NCH = 160
_EP = _CW * _NCH
_NP = 10240
_NBLK = 10
_BN = _NP // _NBLK

_BE = 2560


def _silu(v):
    return v * jax.nn.sigmoid(v)


def _embed_body(xt_ref, w1_ref, b1_ref, w2_ref, b2_ref, o_ref, ot_ref):
    t = _silu(lax.dot_general(xt_ref[...], w1_ref[...], (((0,), (0,)), ((), ())),
                              preferred_element_type=jnp.float32)
              + b1_ref[...])
    h = jnp.dot(t, w2_ref[...], preferred_element_type=jnp.float32) + b2_ref[...]
    o_ref[pl.ds(0, N), :] = h
    ot_ref[:, :, pl.ds(0, N)] = jnp.transpose(h).reshape(_NW, _FPT, N)


def _embed(xt, aW1, ab1, aW2, ab2):
    full = lambda shp: pl.BlockSpec(shp, lambda: tuple(0 for _ in shp))
    return pl.pallas_call(
        _embed_body,
        in_specs=[
            full((AF, N)),
            full((AF, H)),
            full((1, H)),
            full((H, H)),
            full((1, H)),
        ],
        out_specs=[full((_NP, H)), full((_NW, _FPT, _NP))],
        out_shape=[jax.ShapeDtypeStruct((_NP, H), jnp.float32),
                   jax.ShapeDtypeStruct((_NW, _FPT, _NP), jnp.float32)],
    )(xt, aW1, ab1.reshape(1, H), aW2, ab2.reshape(1, H))


def _edge_body(eat_ref, w1_ref, b1_ref, w2_ref, b2_ref, o_ref):
    tt = _silu(lax.dot_general(w1_ref[0], eat_ref[...], (((0,), (0,)), ((), ())),
                               preferred_element_type=jnp.float32)
               + b1_ref[0])
    wt = (lax.dot_general(w2_ref[0], tt, (((0,), (0,)), ((), ())),
                          preferred_element_type=jnp.float32)
          + b2_ref[0])
    o_ref[0] = wt.reshape(_NW, _FPT, _BE)


def _edge_filters(eat, eW1l, eb1l, eW2l, eb2l):
    nblk = E // _BE
    return pl.pallas_call(
        _edge_body,
        grid=(nblk,),
        in_specs=[
            pl.BlockSpec((EF, _BE), lambda i: (0, i)),
            pl.BlockSpec((1, EF, H), lambda i: (0, 0, 0)),
            pl.BlockSpec((1, H, 1), lambda i: (0, 0, 0)),
            pl.BlockSpec((1, H, H), lambda i: (0, 0, 0)),
            pl.BlockSpec((1, H, 1), lambda i: (0, 0, 0)),
        ],
        out_specs=pl.BlockSpec((1, _NW, _FPT, _BE), lambda i: (0, 0, 0, i)),
        out_shape=jax.ShapeDtypeStruct((1, _NW, _FPT, _EP), jnp.float32),
    )(eat, eW1l.reshape(1, EF, H), eb1l.reshape(1, H, 1),
      eW2l.reshape(1, H, H), eb2l.reshape(1, H, 1))


def _make_scatter(layer):
    mesh = plsc.VectorSubcoreMesh(core_axis_name="c", subcore_axis_name="s")

    @functools.partial(
        pl.kernel,
        out_type=jax.ShapeDtypeStruct((_NW, _FPT, _NP), jnp.float32),
        mesh=mesh,
        scratch_types=[
            pltpu.VMEM((_FPT, _NP), jnp.float32),
            pltpu.VMEM((_FPT, _NP), jnp.float32),
            pltpu.VMEM((_FPT, _CW), jnp.float32),
            pltpu.VMEM((_FPT, _CW), jnp.float32),
            pltpu.VMEM((_CW,), jnp.int32),
            pltpu.VMEM((_CW,), jnp.int32),
            pltpu.SemaphoreType.DMA,
            pltpu.SemaphoreType.DMA,
        ],
        compiler_params=pltpu.CompilerParams(needs_layout_passes=False),
    )
    def scatter(ht_hbm, wallt_hbm, sd_hbm, out_hbm,
                hcol, acc, w0, w1, s0, s1, sem0, sem1):
        c = lax.axis_index("c")
        s = lax.axis_index("s")
        t = c * _NTILE + s
        wv = [w0, w1]
        sv = [s0, s1]
        sem = [sem0, sem1]

        pltpu.sync_copy(ht_hbm.at[t], hcol)

        @plsc.parallel_loop(0, _NP // 16, 1, unroll=4)
        def zero(k):
            for f in range(_FPT):
                acc[f, pl.ds(k * 16, 16)] = jnp.zeros((16,), jnp.float32)

        def issue_loads(j, b):
            base = j * _CW
            pltpu.async_copy(wallt_hbm.at[0, t, :, pl.ds(base, _CW)],
                             wv[b], sem[b])
            pltpu.async_copy(sd_hbm.at[pl.ds(base, _CW)], sv[b], sem[b])

        fvec = [jnp.full((16,), f, jnp.int32) for f in range(_FPT)]

        def do_chunk(j, b):
            pltpu.make_async_copy(wallt_hbm.at[0, t, :, pl.ds(0, _CW)],
                                  wv[b], sem[b]).wait()
            pltpu.make_async_copy(sd_hbm.at[pl.ds(0, _CW)], sv[b], sem[b]).wait()

            @plsc.parallel_loop(0, _CW // 16, 1, unroll=4)
            def grp(g):
                packed = sv[b][pl.ds(g * 16, 16)]
                sidx = lax.shift_right_logical(packed, 14)
                didx = lax.bitwise_and(packed, 16383)
                for f in range(_FPT):
                    vals = plsc.load_gather(hcol, [fvec[f], sidx])
                    w = wv[b][f, pl.ds(g * 16, 16)]
                    plsc.addupdate_scatter(acc, [fvec[f], didx], vals * w)

            @pl.when(j + 2 < _NCH)
            def _():
                issue_loads(j + 2, b)

        issue_loads(0, 0)
        issue_loads(1, 1)
        do_chunk(0, 0)
        do_chunk(1, 1)

        def round_body(r, _):
            do_chunk(2 * r, 0)
            do_chunk(2 * r + 1, 1)
            return 0
        lax.fori_loop(1, _NCH // 2, round_body, 0)

        pltpu.sync_copy(acc, out_hbm.at[t])

    return scatter


def _node_body(p_ref, h_ref, w1_ref, b1_ref, w2_ref, b2_ref, g_ref, bt_ref,
               o_ref, ot_ref):
    agg = jnp.transpose(p_ref[...].reshape(H, _BN))
    t = _silu(jnp.dot(agg, w1_ref[...], preferred_element_type=jnp.float32)
              + b1_ref[...])
    out = jnp.dot(t, w2_ref[...], preferred_element_type=jnp.float32) + b2_ref[...]
    z = h_ref[...] + out
    mu = jnp.mean(z, axis=-1, keepdims=True)
    zc = z - mu
    var = jnp.mean(zc * zc, axis=-1, keepdims=True)
    hn = zc / jnp.sqrt(var + 1e-5) * g_ref[...] + bt_ref[...]
    o_ref[...] = hn
    ot_ref[...] = jnp.transpose(hn).reshape(_NW, _FPT, _BN)


def _node(partials, h, nW1l, nb1l, nW2l, nb2l, lngl, lnbl):
    return pl.pallas_call(
        _node_body,
        grid=(_NBLK,),
        in_specs=[
            pl.BlockSpec((_NW, _FPT, _BN), lambda i: (0, 0, i)),
            pl.BlockSpec((_BN, H), lambda i: (i, 0)),
            pl.BlockSpec((H, H), lambda i: (0, 0)),
            pl.BlockSpec((1, H), lambda i: (0, 0)),
            pl.BlockSpec((H, H), lambda i: (0, 0)),
            pl.BlockSpec((1, H), lambda i: (0, 0)),
            pl.BlockSpec((1, H), lambda i: (0, 0)),
            pl.BlockSpec((1, H), lambda i: (0, 0)),
        ],
        out_specs=[pl.BlockSpec((_BN, H), lambda i: (i, 0)),
                   pl.BlockSpec((_NW, _FPT, _BN), lambda i: (0, 0, i))],
        out_shape=[jax.ShapeDtypeStruct((_NP, H), jnp.float32),
                   jax.ShapeDtypeStruct((_NW, _FPT, _NP), jnp.float32)],
    )(partials, h, nW1l, nb1l.reshape(1, H), nW2l, nb2l.reshape(1, H),
      lngl.reshape(1, H), lnbl.reshape(1, H))


def _pool_body(h_ref, b_ref, gf_ref, gW_ref, gb_ref, oW1a_ref, oW1b_ref,
               ob1_ref, oW2_ref, ob2_ref, oW3_ref, ob3_ref, o_ref,
               sums_ref, cnts_ref):
    i = pl.program_id(0)

    @pl.when(i == 0)
    def _init():
        sums_ref[...] = jnp.zeros((B, H), jnp.float32)
        cnts_ref[...] = jnp.zeros((B, H), jnp.float32)

    bvec = b_ref[...]
    oh = (bvec == lax.broadcasted_iota(jnp.int32, (_BN, B), 1)).astype(jnp.float32)
    sums_ref[...] += lax.dot_general(oh, h_ref[...], (((0,), (0,)), ((), ())),
                                     preferred_element_type=jnp.float32)
    cnts_ref[...] += lax.dot_general(oh, jnp.ones((_BN, H), jnp.float32),
                                     (((0,), (0,)), ((), ())),
                                     preferred_element_type=jnp.float32)

    @pl.when(i == _NBLK - 1)
    def _head():
        repr_ = sums_ref[...] / jnp.maximum(cnts_ref[...], 1.0)
        gp = _silu(jnp.dot(gf_ref[...], gW_ref[...],
                           preferred_element_type=jnp.float32) + gb_ref[...])
        h1 = _silu(jnp.dot(repr_, oW1a_ref[...], preferred_element_type=jnp.float32)
                   + jnp.dot(gp, oW1b_ref[...], preferred_element_type=jnp.float32)
                   + ob1_ref[...])
        h2 = _silu(jnp.dot(h1, oW2_ref[...], preferred_element_type=jnp.float32)
                   + ob2_ref[...])
        o_ref[...] = (jnp.dot(h2, oW3_ref[...], preferred_element_type=jnp.float32)
                      + ob3_ref[...])


def _pool_head(h, batch2d, gf, gW, gb, oW1, ob1, oW2, ob2, oW3, ob3):
    full = lambda shp: pl.BlockSpec(shp, lambda i: tuple(0 for _ in shp))
    return pl.pallas_call(
        _pool_body,
        grid=(_NBLK,),
        in_specs=[
            pl.BlockSpec((_BN, H), lambda i: (i, 0)),
            pl.BlockSpec((_BN, 1), lambda i: (i, 0)),
            full((B, GF)),
            full((GF, H)),
            full((1, H)),
            full((H, H)),
            full((H, H)),
            full((1, H)),
            full((H, H // 2)),
            full((1, H // 2)),
            full((H // 2, NT)),
            full((1, NT)),
        ],
        out_specs=full((B, NT)),
        out_shape=jax.ShapeDtypeStruct((B, NT), jnp.float32),
        scratch_shapes=[
            pltpu.VMEM((B, H), jnp.float32),
            pltpu.VMEM((B, H), jnp.float32),
        ],
    )(h, batch2d, gf, gW, gb.reshape(1, H), oW1[:H], oW1[H:],
      ob1.reshape(1, H), oW2, ob2.reshape(1, H // 2), oW3, ob3.reshape(1, NT))


def kernel(x, edge_index, edge_attr, global_features, batch,
           aW1, ab1, aW2, ab2,
           eW1, eb1, eW2, eb2, nW1, nb1, nW2, nb2, lng, lnb,
           gW, gb, oW1, ob1, oW2, ob2, oW3, ob3):
    pad = _EP - E
    src = jnp.concatenate([edge_index[0], jnp.zeros((pad,), jnp.int32)])
    dst = jnp.concatenate([edge_index[1], jnp.full((pad,), N, jnp.int32)])
    sd = src * 16384 + dst
    batchp = jnp.concatenate([batch, jnp.full((_NP - N,), B, jnp.int32)])
    h, ht = _embed(x.T, aW1, ab1, aW2, ab2)
    eat = edge_attr.T
    wallts = [_edge_filters(eat, eW1[l], eb1[l], eW2[l], eb2[l])
              for l in range(L)]
    for l in range(L):
        aggt = _make_scatter(l)(ht, wallts[l], sd)
        h, ht = _node(aggt, h, nW1[l], nb1[l], nW2[l], nb2[l], lng[l], lnb[l])
    return _pool_head(h, batchp.reshape(_NP, 1), global_features, gW, gb,
                      oW1, ob1, oW2, ob2, oW3, ob3)
